# Initial kernel scaffold; baseline (speedup 1.0000x reference)
import jax
import jax.numpy as jnp
from jax.experimental import pallas as pl


def kernel(item_fea, emb_publisher):
    raise NotImplementedError("write your pallas kernel here")


# speedup vs baseline: 1.0573x; 1.0573x over previous
import jax
import jax.numpy as jnp
from jax import lax
from jax.experimental import pallas as pl
from jax.experimental.pallas import tpu as pltpu
from jax.experimental.pallas import tpu_sc as plsc

NUM_PUBLISHER = 100000
EMBED_DIM = 128
BATCH = 16384
N_FEA = 4

_NC = 2
_NS = 16
_L = 16
_NW = _NC * _NS
_B_PER_W = BATCH // _NW
_CHUNK = 128
_NCH = _B_PER_W // _CHUNK


def _gather_body(fea_hbm, table_hbm, out_hbm, fidx_v, idx_v, rows_v, sem):
    wid = lax.axis_index("s") * _NC + lax.axis_index("c")
    base = wid * _B_PER_W

    lanes = lax.iota(jnp.int32, _L) * N_FEA

    def build(j, carry):
        fidx_v[pl.ds(j * _L, _L)] = (base + j * _L) * N_FEA + lanes
        return carry

    lax.fori_loop(0, _B_PER_W // _L, build, 0)

    copies = [
        pltpu.async_copy(
            fea_hbm.at[fidx_v.at[pl.ds(c * _CHUNK, _CHUNK)]],
            idx_v.at[pl.ds(c * _CHUNK, _CHUNK)],
            sem,
        )
        for c in range(_NCH)
    ]
    for cp in copies:
        cp.wait()

    copies = [
        pltpu.async_copy(
            table_hbm.at[idx_v.at[pl.ds(c * _CHUNK, _CHUNK)]],
            rows_v.at[pl.ds(c * _CHUNK, _CHUNK)],
            sem,
        )
        for c in range(_NCH)
    ]
    for cp in copies:
        cp.wait()

    pltpu.sync_copy(rows_v, out_hbm.at[pl.ds(base, _B_PER_W)])


def kernel(item_fea, emb_publisher):
    mesh = plsc.VectorSubcoreMesh(core_axis_name="c", subcore_axis_name="s")
    k = pl.kernel(
        _gather_body,
        out_type=jax.ShapeDtypeStruct((BATCH, EMBED_DIM), jnp.float32),
        mesh=mesh,
        scratch_types=[
            pltpu.VMEM((_B_PER_W,), jnp.int32),
            pltpu.VMEM((_B_PER_W,), jnp.int32),
            pltpu.VMEM((_B_PER_W, EMBED_DIM), jnp.float32),
            pltpu.SemaphoreType.DMA,
        ],
    )
    return k(item_fea.reshape(-1), emb_publisher)
